# Initial kernel scaffold; baseline (speedup 1.0000x reference)
import jax
import jax.numpy as jnp
from jax.experimental import pallas as pl


def kernel(x, pos, batch, edge_index_3rd, num_nodes_per_graph, W_in, b_in, Wg1, bg1, Wg2, bg2, Wmsg, Wupd, bupd, W_out, b_out, Wd1, bd1, Wd2, bd2, Wd3, bd3):
    raise NotImplementedError("write your pallas kernel here")


# speedup vs baseline: 1.6018x; 1.6018x over previous
import functools

import jax
import jax.numpy as jnp
from jax import lax
from jax.experimental import pallas as pl
from jax.experimental.pallas import tpu as pltpu
from jax.experimental.pallas import tpu_sc as plsc

N = 50000
E = 800000
C_IN = 16
H = 64
LATENT = 32
T = 3
B = 64
CUTOFF = 10.0
EPS = 1e-8

NC = 2
NS = 16

NP = 50176
EP = 802816
KB = 128
NBK = 512
NB = NP // NBK
EBK = 512
STRIPE = NP // NS

_mesh = plsc.VectorSubcoreMesh(core_axis_name="c", subcore_axis_name="s")


def _posgather_body(posp, eidx, out, idxv, rows, sem):
  c = lax.axis_index("c")
  s = lax.axis_index("s")
  wid = s * NC + c
  per_tile = EP // (NC * NS)
  nblk = per_tile // KB
  base0 = wid * per_tile
  for p in range(4):
    def blk(b, carry, p=p):
      base = base0 + b * KB
      pltpu.sync_copy(eidx.at[p, pl.ds(base, KB)], idxv)
      pltpu.async_copy(posp.at[idxv], rows, sem).wait()
      pltpu.sync_copy(rows, out.at[p, pl.ds(base, KB), :])
      return carry
    lax.fori_loop(0, nblk, blk, 0)


_sc_params = pltpu.CompilerParams(use_tc_tiling_on_sc=False)

_posgather = pl.kernel(
    _posgather_body,
    out_type=jax.ShapeDtypeStruct((4, EP, 16), jnp.float32),
    mesh=_mesh,
    scratch_types=[
        pltpu.VMEM((KB,), jnp.int32),
        pltpu.VMEM((KB, 16), jnp.float32),
        pltpu.SemaphoreType.DMA,
    ],
    compiler_params=_sc_params,
)


def _msg_body(t, hm0, hm1, ge3, iidx, jidx, zeros, agg0, agg1,
              ibuf, jbuf, rows, gbuf, sem, aggsh):
  c = lax.axis_index("c")
  s = lax.axis_index("s")
  per_sub = EP // NS
  nblk = per_sub // KB

  def run_half(ci, hm, agg):
    pltpu.sync_copy(zeros.at[pl.ds(s * STRIPE, STRIPE), :],
                    aggsh.at[pl.ds(s * STRIPE, STRIPE), :])
    plsc.subcore_barrier()

    def blk(b, carry):
      base = s * per_sub + b * KB
      pltpu.sync_copy(iidx.at[pl.ds(base, KB)], ibuf)
      pltpu.sync_copy(jidx.at[pl.ds(base, KB)], jbuf)
      pltpu.async_copy(hm.at[jbuf], rows, sem).wait()
      pltpu.sync_copy(ge3.at[t, ci, pl.ds(base, KB), :], gbuf)

      def mul(r, carry2):
        rows[r, pl.ds(0, 16)] = rows[r, pl.ds(0, 16)] * gbuf[r, pl.ds(0, 16)]
        rows[r, pl.ds(16, 16)] = rows[r, pl.ds(16, 16)] * gbuf[r, pl.ds(16, 16)]
        return carry2
      lax.fori_loop(0, KB, mul, 0)
      pltpu.sync_copy(rows, aggsh.at[ibuf], add=True)
      return carry
    lax.fori_loop(0, nblk, blk, 0)
    plsc.subcore_barrier()
    pltpu.sync_copy(aggsh.at[pl.ds(s * STRIPE, STRIPE), :],
                    agg.at[pl.ds(s * STRIPE, STRIPE), :])

  @pl.when(c == 0)
  def _():
    run_half(0, hm0, agg0)

  @pl.when(c == 1)
  def _():
    run_half(1, hm1, agg1)


def _make_msg(t):
  return pl.kernel(
      functools.partial(_msg_body, t),
      out_type=(jax.ShapeDtypeStruct((NP, 32), jnp.float32),
                jax.ShapeDtypeStruct((NP, 32), jnp.float32)),
      mesh=_mesh,
      scratch_types=[
          pltpu.VMEM((KB,), jnp.int32),
          pltpu.VMEM((KB,), jnp.int32),
          pltpu.VMEM((KB, 32), jnp.float32),
          pltpu.VMEM((KB, 32), jnp.float32),
          pltpu.SemaphoreType.DMA,
          pltpu.VMEM_SHARED((NP, 32), jnp.float32),
      ],
      compiler_params=_sc_params,
  )


_msg_calls = [_make_msg(t) for t in range(T)]


def _geo_body(gath_ref, wg1_ref, bg1_ref, wg2_ref, bg2_ref, out_ref):
  b = pl.program_id(0)
  a0 = gath_ref[0]
  a1 = gath_ref[1]
  a2 = gath_ref[2]
  a3 = gath_ref[3]

  def comps(v):
    return v[:, 0:1], v[:, 1:2], v[:, 2:3]

  v1x, v1y, v1z = comps(a0 - a1)
  v2x, v2y, v2z = comps(a1 - a2)
  v3x, v3y, v3z = comps(a2 - a3)
  d1 = jnp.sqrt(v1x * v1x + v1y * v1y + v1z * v1z + EPS)
  d2 = jnp.sqrt(v2x * v2x + v2y * v2y + v2z * v2z + EPS)
  d3 = jnp.sqrt(v3x * v3x + v3y * v3y + v3z * v3z + EPS)
  cos_a = (v1x * v2x + v1y * v2y + v1z * v2z) / (d1 * d2)
  cos_b = (v2x * v3x + v2y * v3y + v2z * v3z) / (d2 * d3)
  n1x = v1y * v2z - v1z * v2y
  n1y = v1z * v2x - v1x * v2z
  n1z = v1x * v2y - v1y * v2x
  n2x = v2y * v3z - v2z * v3y
  n2y = v2z * v3x - v2x * v3z
  n2z = v2x * v3y - v2y * v3x
  n1sq = n1x * n1x + n1y * n1y + n1z * n1z
  n2sq = n2x * n2x + n2y * n2y + n2z * n2z
  cos_t = (n1x * n2x + n1y * n2y + n1z * n2z) / jnp.sqrt(
      (n1sq + EPS) * (n2sq + EPS))
  geo = jnp.concatenate([d1, d2, d3, cos_a, cos_b, cos_t], axis=1)
  dc = jnp.minimum(d1, CUTOFF)
  env = 0.5 * (jnp.cos(jnp.pi * dc / CUTOFF) + 1.0) * (d1 < CUTOFF)
  pos_idx = b * EBK + lax.broadcasted_iota(jnp.int32, (EBK, 1), 0)
  env = env * (pos_idx < E)
  for t in range(T):
    gh = jnp.maximum(
        jnp.dot(geo, wg1_ref[t], preferred_element_type=jnp.float32)
        + bg1_ref[t][None, :], 0.0)
    gh = jnp.dot(gh, wg2_ref[t], preferred_element_type=jnp.float32) \
        + bg2_ref[t][None, :]
    ge = gh * env
    out_ref[t, 0] = ge[:, 0:32]
    out_ref[t, 1] = ge[:, 32:64]


def _geo_call(gath, Wg1, bg1, Wg2, bg2):
  return pl.pallas_call(
      _geo_body,
      grid=(EP // EBK,),
      in_specs=[
          pl.BlockSpec((4, EBK, 16), lambda b: (0, b, 0)),
          pl.BlockSpec((T, 6, H), lambda b: (0, 0, 0)),
          pl.BlockSpec((T, H), lambda b: (0, 0)),
          pl.BlockSpec((T, H, H), lambda b: (0, 0, 0)),
          pl.BlockSpec((T, H), lambda b: (0, 0)),
      ],
      out_specs=pl.BlockSpec((T, 2, EBK, 32), lambda b: (0, 0, b, 0)),
      out_shape=jax.ShapeDtypeStruct((T, 2, EP, 32), jnp.float32),
  )(gath, Wg1, bg1, Wg2, bg2)


def _init_body(x_ref, win_ref, bin_ref, wm_ref, h_ref, hm_ref):
  h = jnp.dot(x_ref[...], win_ref[...],
              preferred_element_type=jnp.float32) + bin_ref[...][None, :]
  h_ref[...] = h
  hm = jnp.dot(h, wm_ref[...], preferred_element_type=jnp.float32)
  hm_ref[0] = hm[:, 0:32]
  hm_ref[1] = hm[:, 32:64]


def _init_call(xp, W_in, b_in, Wmsg0):
  return pl.pallas_call(
      _init_body,
      grid=(NB,),
      in_specs=[
          pl.BlockSpec((NBK, C_IN), lambda b: (b, 0)),
          pl.BlockSpec((C_IN, H), lambda b: (0, 0)),
          pl.BlockSpec((H,), lambda b: (0,)),
          pl.BlockSpec((H, H), lambda b: (0, 0)),
      ],
      out_specs=(pl.BlockSpec((NBK, H), lambda b: (b, 0)),
                 pl.BlockSpec((2, NBK, 32), lambda b: (0, b, 0))),
      out_shape=(jax.ShapeDtypeStruct((NP, H), jnp.float32),
                 jax.ShapeDtypeStruct((2, NP, 32), jnp.float32)),
  )(xp, W_in, b_in, Wmsg0)


def _upd_body(has_next, h_ref, a0_ref, a1_ref, wu_ref, bu_ref, wm_ref,
              h_out, hm_out):
  agg = jnp.concatenate([a0_ref[...], a1_ref[...]], axis=1)
  h = h_ref[...] + jnp.maximum(
      jnp.dot(agg, wu_ref[...], preferred_element_type=jnp.float32)
      + bu_ref[...][None, :], 0.0)
  h_out[...] = h
  if has_next:
    hm = jnp.dot(h, wm_ref[...], preferred_element_type=jnp.float32)
    hm_out[0] = hm[:, 0:32]
    hm_out[1] = hm[:, 32:64]


def _upd_call(h, agg0, agg1, Wupd_t, bupd_t, Wmsg_next, has_next):
  return pl.pallas_call(
      functools.partial(_upd_body, has_next),
      grid=(NB,),
      in_specs=[
          pl.BlockSpec((NBK, H), lambda b: (b, 0)),
          pl.BlockSpec((NBK, 32), lambda b: (b, 0)),
          pl.BlockSpec((NBK, 32), lambda b: (b, 0)),
          pl.BlockSpec((H, H), lambda b: (0, 0)),
          pl.BlockSpec((H,), lambda b: (0,)),
          pl.BlockSpec((H, H), lambda b: (0, 0)),
      ],
      out_specs=(pl.BlockSpec((NBK, H), lambda b: (b, 0)),
                 pl.BlockSpec((2, NBK, 32), lambda b: (0, b, 0))),
      out_shape=(jax.ShapeDtypeStruct((NP, H), jnp.float32),
                 jax.ShapeDtypeStruct((2, NP, 32), jnp.float32)),
  )(h, agg0, agg1, Wupd_t, bupd_t, Wmsg_next)


def _readout_body(h_ref, batch_ref, wo_ref, bo_ref, zg_ref, z_ref):
  b = pl.program_id(0)

  @pl.when(b == 0)
  def _():
    zg_ref[...] = jnp.zeros_like(zg_ref)

  bb = batch_ref[0, 0, :]
  pos_idx = b * NBK + lax.broadcasted_iota(jnp.int32, (NBK, B), 0)
  oh = jnp.where(
      (bb[:, None] == lax.broadcasted_iota(jnp.int32, (NBK, B), 1))
      & (pos_idx < N), 1.0, 0.0)
  zg_ref[...] += lax.dot_general(
      oh, h_ref[...], (((0,), (0,)), ((), ())),
      preferred_element_type=jnp.float32)

  @pl.when(b == NB - 1)
  def _():
    z_ref[...] = jnp.dot(zg_ref[...], wo_ref[...],
                         preferred_element_type=jnp.float32) \
        + bo_ref[...][None, :]


def _readout_call(h, batch3d, W_out, b_out):
  return pl.pallas_call(
      _readout_body,
      grid=(NB,),
      in_specs=[
          pl.BlockSpec((NBK, H), lambda b: (b, 0)),
          pl.BlockSpec((1, 1, NBK), lambda b: (b, 0, 0)),
          pl.BlockSpec((H, LATENT), lambda b: (0, 0)),
          pl.BlockSpec((LATENT,), lambda b: (0,)),
      ],
      out_specs=(pl.BlockSpec((B, H), lambda b: (0, 0)),
                 pl.BlockSpec((B, LATENT), lambda b: (0, 0))),
      out_shape=(jax.ShapeDtypeStruct((B, H), jnp.float32),
                 jax.ShapeDtypeStruct((B, LATENT), jnp.float32)),
  )(h, batch3d, W_out, b_out)


def _dec_body(batch_ref, z_ref, w1_ref, b1_ref, w2_ref, b2_ref, w3_ref,
              b3_ref, out_ref):
  bb = batch_ref[0, 0, :]
  oh = jnp.where(
      bb[:, None] == lax.broadcasted_iota(jnp.int32, (NBK, B), 1), 1.0, 0.0)
  zx = jnp.dot(oh, z_ref[...], preferred_element_type=jnp.float32)
  hd = jnp.maximum(
      jnp.dot(zx, w1_ref[...], preferred_element_type=jnp.float32)
      + b1_ref[...][None, :], 0.0)
  hd = jnp.maximum(
      jnp.dot(hd, w2_ref[...], preferred_element_type=jnp.float32)
      + b2_ref[...][None, :], 0.0)
  out_ref[...] = jnp.dot(hd, w3_ref[...],
                         preferred_element_type=jnp.float32) \
      + b3_ref[...][None, :]


def _dec_call(batch3d, z, Wd1, bd1, Wd2, bd2, Wd3, bd3):
  return pl.pallas_call(
      _dec_body,
      grid=(NB,),
      in_specs=[
          pl.BlockSpec((1, 1, NBK), lambda b: (b, 0, 0)),
          pl.BlockSpec((B, LATENT), lambda b: (0, 0)),
          pl.BlockSpec((LATENT, H), lambda b: (0, 0)),
          pl.BlockSpec((H,), lambda b: (0,)),
          pl.BlockSpec((H, 2 * H), lambda b: (0, 0)),
          pl.BlockSpec((2 * H,), lambda b: (0,)),
          pl.BlockSpec((2 * H, C_IN), lambda b: (0, 0)),
          pl.BlockSpec((C_IN,), lambda b: (0,)),
      ],
      out_specs=pl.BlockSpec((NBK, C_IN), lambda b: (b, 0)),
      out_shape=jax.ShapeDtypeStruct((NP, C_IN), jnp.float32),
  )(batch3d, z, Wd1, bd1, Wd2, bd2, Wd3, bd3)


def kernel(x, pos, batch, edge_index_3rd, num_nodes_per_graph,
           W_in, b_in, Wg1, bg1, Wg2, bg2, Wmsg, Wupd, bupd,
           W_out, b_out, Wd1, bd1, Wd2, bd2, Wd3, bd3):
  epad = EP - E
  posp = jnp.pad(pos, ((0, 0), (0, 13)))
  eidx = jnp.pad(edge_index_3rd, ((0, 0), (0, epad)))
  iidx = jnp.concatenate(
      [edge_index_3rd[0], jnp.full((epad,), N, dtype=jnp.int32)])
  jidx = jnp.concatenate(
      [edge_index_3rd[1], jnp.zeros((epad,), dtype=jnp.int32)])
  xp = jnp.pad(x, ((0, NP - N), (0, 0)))
  batch3d = jnp.pad(batch, (0, NP - N)).reshape(NB, 1, NBK)
  zeros = jnp.zeros((NP, 32), jnp.float32)

  gath = _posgather(posp, eidx)
  ge3 = _geo_call(gath, Wg1, bg1, Wg2, bg2)
  h, (hm0, hm1) = _init_call(xp, W_in, b_in, Wmsg[0])
  for t in range(T):
    agg0, agg1 = _msg_calls[t](hm0, hm1, ge3, iidx, jidx, zeros)
    has_next = t + 1 < T
    wm_next = Wmsg[t + 1] if has_next else Wmsg[0]
    h, (hm0, hm1) = _upd_call(h, agg0, agg1, Wupd[t], bupd[t], wm_next,
                              has_next)
  zg, z = _readout_call(h, batch3d, W_out, b_out)
  x_recon = _dec_call(batch3d, z, Wd1, bd1, Wd2, bd2, Wd3, bd3)
  return (x_recon[:N], z)


# speedup vs baseline: 2.0187x; 1.2602x over previous
import functools

import jax
import jax.numpy as jnp
from jax import lax
from jax.experimental import pallas as pl
from jax.experimental.pallas import tpu as pltpu
from jax.experimental.pallas import tpu_sc as plsc

N = 50000
E = 800000
C_IN = 16
H = 64
LATENT = 32
T = 3
B = 64
CUTOFF = 10.0
EPS = 1e-8

NC = 2
NS = 16

NP = 50176
EP = 802816
KB = 128
NBK = 512
NB = NP // NBK
EBK = 512
STRIPE = NP // NS

_mesh = plsc.VectorSubcoreMesh(core_axis_name="c", subcore_axis_name="s")


def _posgather_body(posp, eidx3, out, ib4, rows2, semA0, semA1, semW0, semW1):
  c = lax.axis_index("c")
  s = lax.axis_index("s")
  wid = s * NC + c
  per_tile_blk = (EP // (NC * NS)) // KB
  nsupb = per_tile_blk // 4
  semA = (semA0, semA1)
  semW = (semW0, semW1)
  for p in range(4):
    def superblk(q, carry, p=p):
      row0 = wid * per_tile_blk + q * 4
      @pl.when(q > 0)
      def _():
        pltpu.make_async_copy(
            rows2.at[0], out.at[p, pl.ds(0, KB), :], semW[0]).wait()
        pltpu.make_async_copy(
            rows2.at[1], out.at[p, pl.ds(0, KB), :], semW[1]).wait()
      pltpu.sync_copy(eidx3.at[p, pl.ds(row0, 4), :], ib4)
      descs = {}
      def finish(v):
        d = v & 1
        descs[v].wait()
        descs[('w', v)] = pltpu.async_copy(
            rows2.at[d], out.at[p, pl.ds((row0 + v) * KB, KB), :], semW[d])
      for u in range(4):
        d = u & 1
        if u >= 2:
          descs[('w', u - 2)].wait()
        descs[u] = pltpu.async_copy(posp.at[ib4.at[u]], rows2.at[d], semA[d])
        if u >= 1:
          finish(u - 1)
      finish(3)
      return carry
    lax.fori_loop(0, nsupb, superblk, 0)
    pltpu.make_async_copy(
        rows2.at[0], out.at[p, pl.ds(0, KB), :], semW[0]).wait()
    pltpu.make_async_copy(
        rows2.at[1], out.at[p, pl.ds(0, KB), :], semW[1]).wait()


_sc_params = pltpu.CompilerParams(use_tc_tiling_on_sc=False)

_posgather = pl.kernel(
    _posgather_body,
    out_type=jax.ShapeDtypeStruct((4, EP, 16), jnp.float32),
    mesh=_mesh,
    scratch_types=[
        pltpu.VMEM((4, KB), jnp.int32),
        pltpu.VMEM((2, KB, 16), jnp.float32),
        pltpu.SemaphoreType.DMA,
        pltpu.SemaphoreType.DMA,
        pltpu.SemaphoreType.DMA,
        pltpu.SemaphoreType.DMA,
    ],
    compiler_params=_sc_params,
)


NSUP = 8


NSUP = 8


def _msg_body(t, hm0, hm1, ge3, iidx2, jidx2, zeros, agg0, agg1,
              ib0, ib1, ib2, ib3, ib4, ib5, ib6, ib7,
              jb8, rows2, gb2, semA0, semA1, semS0, semS1, aggsh):
  c = lax.axis_index("c")
  s = lax.axis_index("s")
  per_sub_blk = (EP // NS) // KB
  nsupb = per_sub_blk // NSUP
  semA = (semA0, semA1)
  semS = (semS0, semS1)
  ibs = (ib0, ib1, ib2, ib3, ib4, ib5, ib6, ib7)

  def run_half(ci, hm, agg):
    pltpu.sync_copy(zeros.at[pl.ds(s * STRIPE, STRIPE), :],
                    aggsh.at[pl.ds(s * STRIPE, STRIPE), :])
    plsc.subcore_barrier()

    def mul(dv):
      def mrow(r4, carry2):
        for k in range(4):
          r = r4 * 4 + k
          rows2[dv, r, pl.ds(0, 16)] = (
              rows2[dv, r, pl.ds(0, 16)] * gb2[dv, r, pl.ds(0, 16)])
          rows2[dv, r, pl.ds(16, 16)] = (
              rows2[dv, r, pl.ds(16, 16)] * gb2[dv, r, pl.ds(16, 16)])
        return carry2
      lax.fori_loop(0, KB // 4, mrow, 0)

    def superblk(q, carry):
      @pl.when(q > 0)
      def _():
        pltpu.make_async_copy(
            rows2.at[0], aggsh.at[ibs[NSUP - 2]], semS[0]).wait()
        pltpu.make_async_copy(
            rows2.at[1], aggsh.at[ibs[NSUP - 1]], semS[1]).wait()
      row0 = s * per_sub_blk + q * NSUP
      for u in range(NSUP):
        pltpu.sync_copy(iidx2.at[row0 + u], ibs[u])
      pltpu.sync_copy(jidx2.at[pl.ds(row0, NSUP), :], jb8)
      edge0 = row0 * KB
      descs = {}
      def finish(v):
        d = v & 1
        ga, gb = descs[v]
        ga.wait()
        gb.wait()
        mul(d)
        descs[('s', v)] = pltpu.async_copy(
            rows2.at[d], aggsh.at[ibs[v]], semS[d], add=True)
      for u in range(NSUP):
        d = u & 1
        if u >= 2:
          descs[('s', u - 2)].wait()
        ga = pltpu.async_copy(hm.at[jb8.at[u]], rows2.at[d], semA[d])
        gb = pltpu.async_copy(
            ge3.at[t, ci, pl.ds(edge0 + u * KB, KB), :], gb2.at[d], semA[d])
        descs[u] = (ga, gb)
        if u >= 1:
          finish(u - 1)
      finish(NSUP - 1)
      return carry
    lax.fori_loop(0, nsupb, superblk, 0)
    pltpu.make_async_copy(
        rows2.at[0], aggsh.at[ibs[NSUP - 2]], semS[0]).wait()
    pltpu.make_async_copy(
        rows2.at[1], aggsh.at[ibs[NSUP - 1]], semS[1]).wait()
    plsc.subcore_barrier()
    pltpu.sync_copy(aggsh.at[pl.ds(s * STRIPE, STRIPE), :],
                    agg.at[pl.ds(s * STRIPE, STRIPE), :])

  @pl.when(c == 0)
  def _():
    run_half(0, hm0, agg0)

  @pl.when(c == 1)
  def _():
    run_half(1, hm1, agg1)


def _make_msg(t):
  return pl.kernel(
      functools.partial(_msg_body, t),
      out_type=(jax.ShapeDtypeStruct((NP, 32), jnp.float32),
                jax.ShapeDtypeStruct((NP, 32), jnp.float32)),
      mesh=_mesh,
      scratch_types=[
          pltpu.VMEM((KB,), jnp.int32),
          pltpu.VMEM((KB,), jnp.int32),
          pltpu.VMEM((KB,), jnp.int32),
          pltpu.VMEM((KB,), jnp.int32),
          pltpu.VMEM((KB,), jnp.int32),
          pltpu.VMEM((KB,), jnp.int32),
          pltpu.VMEM((KB,), jnp.int32),
          pltpu.VMEM((KB,), jnp.int32),
          pltpu.VMEM((NSUP, KB), jnp.int32),
          pltpu.VMEM((2, KB, 32), jnp.float32),
          pltpu.VMEM((2, KB, 32), jnp.float32),
          pltpu.SemaphoreType.DMA,
          pltpu.SemaphoreType.DMA,
          pltpu.SemaphoreType.DMA,
          pltpu.SemaphoreType.DMA,
          pltpu.VMEM_SHARED((NP, 32), jnp.float32),
      ],
      compiler_params=_sc_params,
  )


_msg_calls = [_make_msg(t) for t in range(T)]


def _geo_body(gath_ref, wg1_ref, bg1_ref, wg2_ref, bg2_ref, out_ref):
  b = pl.program_id(0)
  a0 = gath_ref[0]
  a1 = gath_ref[1]
  a2 = gath_ref[2]
  a3 = gath_ref[3]

  def comps(v):
    return v[:, 0:1], v[:, 1:2], v[:, 2:3]

  v1x, v1y, v1z = comps(a0 - a1)
  v2x, v2y, v2z = comps(a1 - a2)
  v3x, v3y, v3z = comps(a2 - a3)
  d1 = jnp.sqrt(v1x * v1x + v1y * v1y + v1z * v1z + EPS)
  d2 = jnp.sqrt(v2x * v2x + v2y * v2y + v2z * v2z + EPS)
  d3 = jnp.sqrt(v3x * v3x + v3y * v3y + v3z * v3z + EPS)
  cos_a = (v1x * v2x + v1y * v2y + v1z * v2z) / (d1 * d2)
  cos_b = (v2x * v3x + v2y * v3y + v2z * v3z) / (d2 * d3)
  n1x = v1y * v2z - v1z * v2y
  n1y = v1z * v2x - v1x * v2z
  n1z = v1x * v2y - v1y * v2x
  n2x = v2y * v3z - v2z * v3y
  n2y = v2z * v3x - v2x * v3z
  n2z = v2x * v3y - v2y * v3x
  n1sq = n1x * n1x + n1y * n1y + n1z * n1z
  n2sq = n2x * n2x + n2y * n2y + n2z * n2z
  cos_t = (n1x * n2x + n1y * n2y + n1z * n2z) / jnp.sqrt(
      (n1sq + EPS) * (n2sq + EPS))
  geo = jnp.concatenate([d1, d2, d3, cos_a, cos_b, cos_t], axis=1)
  dc = jnp.minimum(d1, CUTOFF)
  env = 0.5 * (jnp.cos(jnp.pi * dc / CUTOFF) + 1.0) * (d1 < CUTOFF)
  pos_idx = b * EBK + lax.broadcasted_iota(jnp.int32, (EBK, 1), 0)
  env = env * (pos_idx < E)
  for t in range(T):
    gh = jnp.maximum(
        jnp.dot(geo, wg1_ref[t], preferred_element_type=jnp.float32)
        + bg1_ref[t][None, :], 0.0)
    gh = jnp.dot(gh, wg2_ref[t], preferred_element_type=jnp.float32) \
        + bg2_ref[t][None, :]
    ge = gh * env
    out_ref[t, 0] = ge[:, 0:32]
    out_ref[t, 1] = ge[:, 32:64]


def _geo_call(gath, Wg1, bg1, Wg2, bg2):
  return pl.pallas_call(
      _geo_body,
      grid=(EP // EBK,),
      in_specs=[
          pl.BlockSpec((4, EBK, 16), lambda b: (0, b, 0)),
          pl.BlockSpec((T, 6, H), lambda b: (0, 0, 0)),
          pl.BlockSpec((T, H), lambda b: (0, 0)),
          pl.BlockSpec((T, H, H), lambda b: (0, 0, 0)),
          pl.BlockSpec((T, H), lambda b: (0, 0)),
      ],
      out_specs=pl.BlockSpec((T, 2, EBK, 32), lambda b: (0, 0, b, 0)),
      out_shape=jax.ShapeDtypeStruct((T, 2, EP, 32), jnp.float32),
  )(gath, Wg1, bg1, Wg2, bg2)


def _init_body(x_ref, win_ref, bin_ref, wm_ref, h_ref, hm_ref):
  h = jnp.dot(x_ref[...], win_ref[...],
              preferred_element_type=jnp.float32) + bin_ref[...][None, :]
  h_ref[...] = h
  hm = jnp.dot(h, wm_ref[...], preferred_element_type=jnp.float32)
  hm_ref[0] = hm[:, 0:32]
  hm_ref[1] = hm[:, 32:64]


def _init_call(xp, W_in, b_in, Wmsg0):
  return pl.pallas_call(
      _init_body,
      grid=(NB,),
      in_specs=[
          pl.BlockSpec((NBK, C_IN), lambda b: (b, 0)),
          pl.BlockSpec((C_IN, H), lambda b: (0, 0)),
          pl.BlockSpec((H,), lambda b: (0,)),
          pl.BlockSpec((H, H), lambda b: (0, 0)),
      ],
      out_specs=(pl.BlockSpec((NBK, H), lambda b: (b, 0)),
                 pl.BlockSpec((2, NBK, 32), lambda b: (0, b, 0))),
      out_shape=(jax.ShapeDtypeStruct((NP, H), jnp.float32),
                 jax.ShapeDtypeStruct((2, NP, 32), jnp.float32)),
  )(xp, W_in, b_in, Wmsg0)


def _upd_body(has_next, h_ref, a0_ref, a1_ref, wu_ref, bu_ref, wm_ref,
              h_out, hm_out):
  agg = jnp.concatenate([a0_ref[...], a1_ref[...]], axis=1)
  h = h_ref[...] + jnp.maximum(
      jnp.dot(agg, wu_ref[...], preferred_element_type=jnp.float32)
      + bu_ref[...][None, :], 0.0)
  h_out[...] = h
  if has_next:
    hm = jnp.dot(h, wm_ref[...], preferred_element_type=jnp.float32)
    hm_out[0] = hm[:, 0:32]
    hm_out[1] = hm[:, 32:64]


def _upd_call(h, agg0, agg1, Wupd_t, bupd_t, Wmsg_next, has_next):
  return pl.pallas_call(
      functools.partial(_upd_body, has_next),
      grid=(NB,),
      in_specs=[
          pl.BlockSpec((NBK, H), lambda b: (b, 0)),
          pl.BlockSpec((NBK, 32), lambda b: (b, 0)),
          pl.BlockSpec((NBK, 32), lambda b: (b, 0)),
          pl.BlockSpec((H, H), lambda b: (0, 0)),
          pl.BlockSpec((H,), lambda b: (0,)),
          pl.BlockSpec((H, H), lambda b: (0, 0)),
      ],
      out_specs=(pl.BlockSpec((NBK, H), lambda b: (b, 0)),
                 pl.BlockSpec((2, NBK, 32), lambda b: (0, b, 0))),
      out_shape=(jax.ShapeDtypeStruct((NP, H), jnp.float32),
                 jax.ShapeDtypeStruct((2, NP, 32), jnp.float32)),
  )(h, agg0, agg1, Wupd_t, bupd_t, Wmsg_next)


def _readout_body(h_ref, batch_ref, wo_ref, bo_ref, zg_ref, z_ref):
  b = pl.program_id(0)

  @pl.when(b == 0)
  def _():
    zg_ref[...] = jnp.zeros_like(zg_ref)

  bb = batch_ref[0, 0, :]
  pos_idx = b * NBK + lax.broadcasted_iota(jnp.int32, (NBK, B), 0)
  oh = jnp.where(
      (bb[:, None] == lax.broadcasted_iota(jnp.int32, (NBK, B), 1))
      & (pos_idx < N), 1.0, 0.0)
  zg_ref[...] += lax.dot_general(
      oh, h_ref[...], (((0,), (0,)), ((), ())),
      preferred_element_type=jnp.float32)

  @pl.when(b == NB - 1)
  def _():
    z_ref[...] = jnp.dot(zg_ref[...], wo_ref[...],
                         preferred_element_type=jnp.float32) \
        + bo_ref[...][None, :]


def _readout_call(h, batch3d, W_out, b_out):
  return pl.pallas_call(
      _readout_body,
      grid=(NB,),
      in_specs=[
          pl.BlockSpec((NBK, H), lambda b: (b, 0)),
          pl.BlockSpec((1, 1, NBK), lambda b: (b, 0, 0)),
          pl.BlockSpec((H, LATENT), lambda b: (0, 0)),
          pl.BlockSpec((LATENT,), lambda b: (0,)),
      ],
      out_specs=(pl.BlockSpec((B, H), lambda b: (0, 0)),
                 pl.BlockSpec((B, LATENT), lambda b: (0, 0))),
      out_shape=(jax.ShapeDtypeStruct((B, H), jnp.float32),
                 jax.ShapeDtypeStruct((B, LATENT), jnp.float32)),
  )(h, batch3d, W_out, b_out)


def _dec_body(batch_ref, z_ref, w1_ref, b1_ref, w2_ref, b2_ref, w3_ref,
              b3_ref, out_ref):
  bb = batch_ref[0, 0, :]
  oh = jnp.where(
      bb[:, None] == lax.broadcasted_iota(jnp.int32, (NBK, B), 1), 1.0, 0.0)
  zx = jnp.dot(oh, z_ref[...], preferred_element_type=jnp.float32)
  hd = jnp.maximum(
      jnp.dot(zx, w1_ref[...], preferred_element_type=jnp.float32)
      + b1_ref[...][None, :], 0.0)
  hd = jnp.maximum(
      jnp.dot(hd, w2_ref[...], preferred_element_type=jnp.float32)
      + b2_ref[...][None, :], 0.0)
  out_ref[...] = jnp.dot(hd, w3_ref[...],
                         preferred_element_type=jnp.float32) \
      + b3_ref[...][None, :]


def _dec_call(batch3d, z, Wd1, bd1, Wd2, bd2, Wd3, bd3):
  return pl.pallas_call(
      _dec_body,
      grid=(NB,),
      in_specs=[
          pl.BlockSpec((1, 1, NBK), lambda b: (b, 0, 0)),
          pl.BlockSpec((B, LATENT), lambda b: (0, 0)),
          pl.BlockSpec((LATENT, H), lambda b: (0, 0)),
          pl.BlockSpec((H,), lambda b: (0,)),
          pl.BlockSpec((H, 2 * H), lambda b: (0, 0)),
          pl.BlockSpec((2 * H,), lambda b: (0,)),
          pl.BlockSpec((2 * H, C_IN), lambda b: (0, 0)),
          pl.BlockSpec((C_IN,), lambda b: (0,)),
      ],
      out_specs=pl.BlockSpec((NBK, C_IN), lambda b: (b, 0)),
      out_shape=jax.ShapeDtypeStruct((NP, C_IN), jnp.float32),
  )(batch3d, z, Wd1, bd1, Wd2, bd2, Wd3, bd3)


def kernel(x, pos, batch, edge_index_3rd, num_nodes_per_graph,
           W_in, b_in, Wg1, bg1, Wg2, bg2, Wmsg, Wupd, bupd,
           W_out, b_out, Wd1, bd1, Wd2, bd2, Wd3, bd3):
  epad = EP - E
  posp = jnp.pad(pos, ((0, 0), (0, 13)))
  eidx = jnp.pad(edge_index_3rd, ((0, 0), (0, epad)))
  iidx = jnp.concatenate(
      [edge_index_3rd[0], jnp.full((epad,), N, dtype=jnp.int32)])
  jidx = jnp.concatenate(
      [edge_index_3rd[1], jnp.zeros((epad,), dtype=jnp.int32)])
  xp = jnp.pad(x, ((0, NP - N), (0, 0)))
  batch3d = jnp.pad(batch, (0, NP - N)).reshape(NB, 1, NBK)
  zeros = jnp.zeros((NP, 32), jnp.float32)

  eidx3 = eidx.reshape(4, EP // KB, KB)
  iidx2 = iidx.reshape(EP // KB, KB)
  jidx2 = jidx.reshape(EP // KB, KB)

  gath = _posgather(posp, eidx3)
  ge3 = _geo_call(gath, Wg1, bg1, Wg2, bg2)
  h, (hm0, hm1) = _init_call(xp, W_in, b_in, Wmsg[0])
  for t in range(T):
    agg0, agg1 = _msg_calls[t](hm0, hm1, ge3, iidx2, jidx2, zeros)
    has_next = t + 1 < T
    wm_next = Wmsg[t + 1] if has_next else Wmsg[0]
    h, (hm0, hm1) = _upd_call(h, agg0, agg1, Wupd[t], bupd[t], wm_next,
                              has_next)
  zg, z = _readout_call(h, batch3d, W_out, b_out)
  x_recon = _dec_call(batch3d, z, Wd1, bd1, Wd2, bd2, Wd3, bd3)
  return (x_recon[:N], z)
